# baseline (speedup 1.0000x reference)
import functools

import jax
import jax.numpy as jnp
from jax import lax
from jax.experimental import pallas as pl
from jax.experimental.pallas import tpu as pltpu
from jax.experimental.pallas import tpu_sc as plsc

N_NODES = 100000
N_EDGES = 6400000

NC = 2
NS = 16
NW = NC * NS

CHUNK = 2048
NB = CHUNK // 128
NCHUNK = N_EDGES // CHUNK
N_EXTRA = NCHUNK - (NCHUNK // NW) * NW
N_MAIN = 96
ACC = 102400
ACC_PER_TILE = ACC // NS
ZCHUNK = 1280


def _gather_chunk(x_v, src_r, vals_r):
    for j in range(NB):
        for i in range(8):
            off = i * 16
            idx = src_r[j, pl.ds(off, 16)]
            vals_r[pl.ds(j * 128 + off, 16)] = plsc.load_gather(x_v, [idx])


def _layer_body(is_first, *refs):
    if is_first:
        (x_hbm, ed_hbm, pout, x_v, src0, src1, dst0, dst1, dst2,
         dst3, vals0, vals1, zbuf, acc_sh, sin0, sin1, ss0, ss1) = refs
    else:
        (p_hbm, w_hbm, ed_hbm, pout, x_v, src0, src1, dst0, dst1,
         dst2, dst3, vals0, vals1, zbuf, wv, acc_sh, sin0, sin1, ss0,
         ss1) = refs

    srcs = [src0, src1]
    dsts = [dst0, dst1, dst2, dst3]
    vals = [vals0, vals1]
    sins = [sin0, sin1]
    sss = [ss0, ss1]

    c = lax.axis_index("c")
    s = lax.axis_index("s")
    w = c * NS + s

    if is_first:
        pltpu.sync_copy(x_hbm, x_v.at[pl.ds(0, N_NODES)])
    else:
        pltpu.sync_copy(w_hbm, wv)
        wvec = wv[pl.ds(0, 16)]
        pltpu.sync_copy(p_hbm.at[pl.ds(0, ACC)], x_v)
        pltpu.async_copy(p_hbm.at[pl.ds(ACC, CHUNK)], vals0, sin0)
        nmerge = ACC // CHUNK

        def merge_iter(m, _):
            for h in range(2):
                j = 2 * m + h
                buf = vals[h]
                pltpu.make_async_copy(p_hbm.at[pl.ds(ACC, CHUNK)], buf,
                                      sin0).wait()
                nxt = j + 1
                if h == 0:
                    pltpu.async_copy(
                        p_hbm.at[pl.ds(ACC + nxt * CHUNK, CHUNK)], vals[1],
                        sin0)
                else:
                    @pl.when(nxt < nmerge)
                    def _():
                        pltpu.async_copy(
                            p_hbm.at[pl.ds(ACC + nxt * CHUNK, CHUNK)],
                            vals[0], sin0)
                base = j * CHUNK
                for i in range(CHUNK // 16):
                    sl = pl.ds(base + i * 16, 16)
                    x_v[sl] = (x_v[sl] + buf[pl.ds(i * 16, 16)]) * wvec
            return 0

        lax.fori_loop(0, nmerge // 2, merge_iter, 0)

    def zbody(i, _):
        zbuf[pl.ds(i * 16, 16)] = jnp.zeros((16,), jnp.float32)
        return 0

    lax.fori_loop(0, ZCHUNK // 16, zbody, 0)
    for t in range(ACC_PER_TILE // ZCHUNK):
        pltpu.async_copy(
            zbuf, acc_sh.at[pl.ds(s * ACC_PER_TILE + t * ZCHUNK, ZCHUNK)],
            sin1)
    for t in range(ACC_PER_TILE // ZCHUNK):
        pltpu.make_async_copy(pout.at[pl.ds(0, ZCHUNK)], zbuf, sin1).wait()

    n_w = jnp.where(w < N_EXTRA, NCHUNK // NW + 1, NCHUNK // NW)

    def cidx(k):
        return w + NW * k

    def issue_in(k, p, q):
        pltpu.async_copy(ed_hbm.at[0, cidx(k)], srcs[p], sins[p])
        pltpu.async_copy(ed_hbm.at[1, cidx(k)], dsts[q], sins[p])

    def wait_in(p):
        pltpu.make_async_copy(ed_hbm.at[0, 0], srcs[p], sins[p]).wait()
        pltpu.make_async_copy(ed_hbm.at[1, 0], dsts[0], sins[p]).wait()

    def fire(p, q):
        for j in range(NB):
            pltpu.async_copy(vals[p].at[pl.ds(j * 128, 128)],
                             acc_sh.at[dsts[q].at[j]], sss[p], add=True)

    def drain_s(p):
        pltpu.make_async_copy(pout.at[pl.ds(0, CHUNK)], vals[p],
                              sss[p]).wait()

    issue_in(0, 0, 0)
    issue_in(1, 1, 1)

    plsc.subcore_barrier()

    def main_iter(m, _):
        for q in range(4):
            p = q % 2
            k = 4 * m + q
            wait_in(p)
            if q < 2:
                @pl.when(m > 0)
                def _():
                    drain_s(p)
            else:
                drain_s(p)
            _gather_chunk(x_v, srcs[p], vals[p])
            fire(p, q)

            @pl.when(k + 2 < n_w)
            def _():
                issue_in(k + 2, p, (q + 2) % 4)
        return 0

    lax.fori_loop(0, N_MAIN // 4, main_iter, 0)

    wait_in(0)
    drain_s(0)
    _gather_chunk(x_v, srcs[0], vals[0])
    fire(0, 0)

    @pl.when(n_w > N_MAIN + 1)
    def _():
        wait_in(1)
        drain_s(1)
        _gather_chunk(x_v, srcs[1], vals[1])
        fire(1, 1)

    drain_s(0)
    drain_s(1)

    plsc.subcore_barrier()

    pltpu.sync_copy(acc_sh.at[pl.ds(s * ACC_PER_TILE, ACC_PER_TILE)],
                    pout.at[pl.ds(c * ACC + s * ACC_PER_TILE, ACC_PER_TILE)])


_MESH = plsc.VectorSubcoreMesh(core_axis_name="c", subcore_axis_name="s",
                               num_cores=NC, num_subcores=NS)

_SC_PARAMS = pltpu.CompilerParams(needs_layout_passes=False)


def _edge_scratch():
    return [
        pltpu.VMEM((ACC,), jnp.float32),
        pltpu.VMEM((NB, 128), jnp.int32),
        pltpu.VMEM((NB, 128), jnp.int32),
        pltpu.VMEM((NB, 128), jnp.int32),
        pltpu.VMEM((NB, 128), jnp.int32),
        pltpu.VMEM((NB, 128), jnp.int32),
        pltpu.VMEM((NB, 128), jnp.int32),
        pltpu.VMEM((CHUNK,), jnp.float32),
        pltpu.VMEM((CHUNK,), jnp.float32),
        pltpu.VMEM((ZCHUNK,), jnp.float32),
    ]


def _sems():
    return [
        pltpu.SemaphoreType.DMA,
        pltpu.SemaphoreType.DMA,
        pltpu.SemaphoreType.DMA,
        pltpu.SemaphoreType.DMA,
    ]


_layer1 = pl.kernel(
    functools.partial(_layer_body, True),
    out_type=jax.ShapeDtypeStruct((NC * ACC,), jnp.float32),
    mesh=_MESH,
    scratch_types=_edge_scratch() + [
        pltpu.VMEM_SHARED((ACC,), jnp.float32),
    ] + _sems(),
    compiler_params=_SC_PARAMS,
)

_layer2 = pl.kernel(
    functools.partial(_layer_body, False),
    out_type=jax.ShapeDtypeStruct((NC * ACC,), jnp.float32),
    mesh=_MESH,
    scratch_types=_edge_scratch() + [
        pltpu.VMEM((16,), jnp.float32),
        pltpu.VMEM_SHARED((ACC,), jnp.float32),
    ] + _sems(),
    compiler_params=_SC_PARAMS,
)


def _merge_body(p_ref, o_ref):
    a = p_ref[...]
    o_ref[...] = a[:ACC // 128] + a[ACC // 128:]


_merge_tc = pl.pallas_call(
    _merge_body,
    out_shape=jax.ShapeDtypeStruct((ACC // 128, 128), jnp.float32),
)


@jax.jit
def kernel(x, edge_index, W1, W2):
    xf = x.reshape(-1)
    ed = edge_index.reshape(2, NCHUNK, NB, 128)
    p1 = _layer1(xf, ed)
    wvec = jnp.full((16,), W1[0, 0] * W2[0, 0], dtype=jnp.float32)
    p2 = _layer2(p1, wvec, ed)
    merged = _merge_tc(p2.reshape(NC * ACC // 128, 128))
    return merged.reshape(-1)[:N_NODES][0::5]


# speedup vs baseline: 1.1636x; 1.1636x over previous
import functools

import jax
import jax.numpy as jnp
from jax import lax
from jax.experimental import pallas as pl
from jax.experimental.pallas import tpu as pltpu
from jax.experimental.pallas import tpu_sc as plsc

N_NODES = 100000
N_EDGES = 6400000

NC = 2
NS = 16
NW = NC * NS

CHUNK = 2048
NB = CHUNK // 128
NCHUNK = N_EDGES // CHUNK
N_EXTRA = NCHUNK - (NCHUNK // NW) * NW
N_MAIN = 96
ACC = 102400
ACC_PER_TILE = ACC // NS
ZCHUNK = 1280


def _gather_chunk(x_v, ebuf, vals_r):
    for j in range(NB):
        for i in range(8):
            off = i * 16
            idx = ebuf[2 * j, pl.ds(off, 16)]
            vals_r[pl.ds(j * 128 + off, 16)] = plsc.load_gather(x_v, [idx])


def _layer_body(is_first, *refs):
    if is_first:
        (x_hbm, ed_hbm, pout, x_v, eb0, eb1, eb2, eb3, vals0, vals1,
         zbuf, acc_sh, sin0, sin1, ss0, ss1) = refs
    else:
        (p_hbm, w_hbm, ed_hbm, pout, x_v, eb0, eb1, eb2, eb3, vals0,
         vals1, zbuf, wv, acc_sh, sin0, sin1, ss0, ss1) = refs

    ebufs = [eb0, eb1, eb2, eb3]
    vals = [vals0, vals1]
    sins = [sin0, sin1]
    sss = [ss0, ss1]

    c = lax.axis_index("c")
    s = lax.axis_index("s")
    w = c * NS + s

    if is_first:
        pltpu.sync_copy(x_hbm, x_v.at[pl.ds(0, N_NODES)])
    else:
        pltpu.sync_copy(w_hbm, wv)
        wvec = wv[pl.ds(0, 16)]
        pltpu.sync_copy(p_hbm.at[pl.ds(0, ACC)], x_v)
        pltpu.async_copy(p_hbm.at[pl.ds(ACC, CHUNK)], vals0, sin0)
        nmerge = ACC // CHUNK

        def merge_iter(m, _):
            for h in range(2):
                j = 2 * m + h
                buf = vals[h]
                pltpu.make_async_copy(p_hbm.at[pl.ds(ACC, CHUNK)], buf,
                                      sin0).wait()
                nxt = j + 1
                if h == 0:
                    pltpu.async_copy(
                        p_hbm.at[pl.ds(ACC + nxt * CHUNK, CHUNK)], vals[1],
                        sin0)
                else:
                    @pl.when(nxt < nmerge)
                    def _():
                        pltpu.async_copy(
                            p_hbm.at[pl.ds(ACC + nxt * CHUNK, CHUNK)],
                            vals[0], sin0)
                base = j * CHUNK
                for i in range(CHUNK // 16):
                    sl = pl.ds(base + i * 16, 16)
                    x_v[sl] = (x_v[sl] + buf[pl.ds(i * 16, 16)]) * wvec
            return 0

        lax.fori_loop(0, nmerge // 2, merge_iter, 0)

    def zbody(i, _):
        zbuf[pl.ds(i * 16, 16)] = jnp.zeros((16,), jnp.float32)
        return 0

    lax.fori_loop(0, ZCHUNK // 16, zbody, 0)
    for t in range(ACC_PER_TILE // ZCHUNK):
        pltpu.async_copy(
            zbuf, acc_sh.at[pl.ds(s * ACC_PER_TILE + t * ZCHUNK, ZCHUNK)],
            sin1)
    for t in range(ACC_PER_TILE // ZCHUNK):
        pltpu.make_async_copy(pout.at[pl.ds(0, ZCHUNK)], zbuf, sin1).wait()

    n_w = jnp.where(w < N_EXTRA, NCHUNK // NW + 1, NCHUNK // NW)

    def cidx(k):
        return w + NW * k

    def issue_in(k, p, q):
        pltpu.async_copy(ed_hbm.at[pl.ds(32 * cidx(k), 32)], ebufs[q],
                         sins[p])

    def wait_in(p):
        pltpu.make_async_copy(ed_hbm.at[pl.ds(0, 32)], ebufs[0],
                              sins[p]).wait()

    def fire(p, q):
        for j in range(NB):
            pltpu.async_copy(vals[p].at[pl.ds(j * 128, 128)],
                             acc_sh.at[ebufs[q].at[2 * j + 1]], sss[p],
                             add=True)

    def drain_s(p):
        pltpu.make_async_copy(pout.at[pl.ds(0, CHUNK)], vals[p],
                              sss[p]).wait()

    issue_in(0, 0, 0)
    issue_in(1, 1, 1)

    plsc.subcore_barrier()

    def main_iter(m, _):
        for q in range(4):
            p = q % 2
            k = 4 * m + q
            wait_in(p)
            if q < 2:
                @pl.when(m > 0)
                def _():
                    drain_s(p)
            else:
                drain_s(p)
            _gather_chunk(x_v, ebufs[q], vals[p])
            fire(p, q)

            @pl.when(k + 2 < n_w)
            def _():
                issue_in(k + 2, p, (q + 2) % 4)
        return 0

    lax.fori_loop(0, N_MAIN // 4, main_iter, 0)

    wait_in(0)
    drain_s(0)
    _gather_chunk(x_v, ebufs[0], vals[0])
    fire(0, 0)

    @pl.when(n_w > N_MAIN + 1)
    def _():
        wait_in(1)
        drain_s(1)
        _gather_chunk(x_v, ebufs[1], vals[1])
        fire(1, 1)

    drain_s(0)
    drain_s(1)

    plsc.subcore_barrier()

    pltpu.sync_copy(acc_sh.at[pl.ds(s * ACC_PER_TILE, ACC_PER_TILE)],
                    pout.at[pl.ds(c * ACC + s * ACC_PER_TILE, ACC_PER_TILE)])


_MESH = plsc.VectorSubcoreMesh(core_axis_name="c", subcore_axis_name="s",
                               num_cores=NC, num_subcores=NS)

_SC_PARAMS = pltpu.CompilerParams(needs_layout_passes=False)


def _edge_scratch():
    return [
        pltpu.VMEM((ACC,), jnp.float32),
        pltpu.VMEM((32, 128), jnp.int32),
        pltpu.VMEM((32, 128), jnp.int32),
        pltpu.VMEM((32, 128), jnp.int32),
        pltpu.VMEM((32, 128), jnp.int32),
        pltpu.VMEM((CHUNK,), jnp.float32),
        pltpu.VMEM((CHUNK,), jnp.float32),
        pltpu.VMEM((ZCHUNK,), jnp.float32),
    ]


def _sems():
    return [
        pltpu.SemaphoreType.DMA,
        pltpu.SemaphoreType.DMA,
        pltpu.SemaphoreType.DMA,
        pltpu.SemaphoreType.DMA,
    ]


_layer1 = pl.kernel(
    functools.partial(_layer_body, True),
    out_type=jax.ShapeDtypeStruct((NC * ACC,), jnp.float32),
    mesh=_MESH,
    scratch_types=_edge_scratch() + [
        pltpu.VMEM_SHARED((ACC,), jnp.float32),
    ] + _sems(),
    compiler_params=_SC_PARAMS,
)

_layer2 = pl.kernel(
    functools.partial(_layer_body, False),
    out_type=jax.ShapeDtypeStruct((NC * ACC,), jnp.float32),
    mesh=_MESH,
    scratch_types=_edge_scratch() + [
        pltpu.VMEM((16,), jnp.float32),
        pltpu.VMEM_SHARED((ACC,), jnp.float32),
    ] + _sems(),
    compiler_params=_SC_PARAMS,
)


def _merge_body(p_ref, o_ref):
    a = p_ref[...]
    o_ref[...] = a[:ACC // 128] + a[ACC // 128:]


_merge_tc = pl.pallas_call(
    _merge_body,
    out_shape=jax.ShapeDtypeStruct((ACC // 128, 128), jnp.float32),
)


@jax.jit
def kernel(x, edge_index, W1, W2):
    xf = x.reshape(-1)
    ed = edge_index.reshape(2, N_EDGES // 128, 128).transpose(1, 0, 2)
    ed = ed.reshape(2 * (N_EDGES // 128), 128)
    p1 = _layer1(xf, ed)
    wvec = jnp.full((16,), W1[0, 0] * W2[0, 0], dtype=jnp.float32)
    p2 = _layer2(p1, wvec, ed)
    merged = _merge_tc(p2.reshape(NC * ACC // 128, 128))
    return merged.reshape(-1)[:N_NODES][0::5]
